# Initial kernel scaffold; baseline (speedup 1.0000x reference)
import jax
import jax.numpy as jnp
from jax.experimental import pallas as pl


def kernel(logit, target):
    raise NotImplementedError("write your pallas kernel here")


# speedup vs baseline: 74.4834x; 74.4834x over previous
import jax
import jax.numpy as jnp
from jax import lax
from jax.experimental import pallas as pl

MPOS = 2.5
MNEG = 0.5
GAMMA = 2.0

CH = 8


def _loss_kernel(xt_ref, tgt_ref, out_ref):
    i = pl.program_id(0)
    t = tgt_ref[0]
    c, nb = xt_ref.shape
    nch = c // CH

    iota8 = lax.broadcasted_iota(jnp.int32, (CH, nb), 0)
    neg = jnp.full((CH, nb), -jnp.inf, dtype=jnp.float32)

    def body(k, carry):
        hi, lo, gacc = carry
        xc = xt_ref[pl.ds(k * CH, CH), :]
        mn = jnp.minimum(hi, xc)
        hi = jnp.maximum(hi, xc)
        lo = jnp.maximum(lo, mn)
        rel = t - k * CH
        gacc = gacc + jnp.where(iota8 == rel, xc, 0.0)
        return hi, lo, gacc

    hi0 = xt_ref[0:CH, :]
    g0 = jnp.where(iota8 == t, hi0, 0.0)
    hi, lo, gacc = lax.fori_loop(1, nch, body, (hi0, neg, g0), unroll=4)

    while hi.shape[0] > 1:
        h = hi.shape[0] // 2
        h1, h2 = hi[:h], hi[h:]
        l1, l2 = lo[:h], lo[h:]
        lo = jnp.maximum(jnp.minimum(h1, h2), jnp.maximum(l1, l2))
        hi = jnp.maximum(h1, h2)
    g = jnp.sum(gacc, axis=0, keepdims=True)

    a = jnp.exp(GAMMA * (MPOS - g))
    v = jnp.where((g == hi) & (lo < hi), lo, hi)
    bv = jnp.exp(GAMMA * (MNEG + v))
    prod = (1.0 + bv) * jnp.where(t != 0, 1.0 + a, 1.0)
    contrib = jnp.sum(jnp.log(prod))

    @pl.when(i == 0)
    def _():
        out_ref[...] = jnp.zeros_like(out_ref)

    out_ref[...] += contrib


def kernel(logit, target):
    b, c = logit.shape
    nb = 1024
    grid = b // nb
    xt = logit.T
    t3 = target.reshape(grid, 1, nb)
    out = pl.pallas_call(
        _loss_kernel,
        grid=(grid,),
        in_specs=[
            pl.BlockSpec((c, nb), lambda i: (0, i)),
            pl.BlockSpec((1, 1, nb), lambda i: (i, 0, 0)),
        ],
        out_specs=pl.BlockSpec((1, 1), lambda i: (0, 0)),
        out_shape=jax.ShapeDtypeStruct((1, 1), jnp.float32),
    )(xt, t3)
    return out[0, 0] / b
